# Initial kernel scaffold; baseline (speedup 1.0000x reference)
import jax
import jax.numpy as jnp
from jax.experimental import pallas as pl


def kernel(p1, p2):
    raise NotImplementedError("write your pallas kernel here")


# speedup vs baseline: 12.8736x; 12.8736x over previous
import jax
import jax.numpy as jnp
from jax.experimental import pallas as pl

K = 16
BQ = 256
P2 = 4096
D = 128


def _knn_body(p1_ref, p2_ref, p2t_ref, idx_ref, dist_ref):
    p1b = p1_ref[0]
    p2b = p2_ref[0]
    p2t = p2t_ref[0]

    inner = jax.lax.dot_general(
        p1b, p2b, (((1,), (1,)), ((), ())),
        preferred_element_type=jnp.float32)

    sq1 = jnp.sum(p1b * p1b, axis=1, keepdims=True)
    sq2 = jnp.sum(p2t * p2t, axis=0, keepdims=True)

    d = sq1 + sq2 - 2.0 * inner

    iota = jax.lax.broadcasted_iota(jnp.int32, (BQ, P2), 1)
    inf = jnp.float32(jnp.inf)
    for k in range(K):
        m = jnp.min(d, axis=1, keepdims=True)
        cand = jnp.where(d == m, iota, jnp.int32(P2))
        j = jnp.min(cand, axis=1, keepdims=True)
        idx_ref[0, :, pl.ds(k, 1)] = j
        dist_ref[0, :, pl.ds(k, 1)] = m
        d = jnp.where(iota == j, inf, d)


def kernel(p1, p2):
    n, p1n, d_ = p1.shape
    grid = (n, p1n // BQ)
    idx, dists = pl.pallas_call(
        _knn_body,
        grid=grid,
        in_specs=[
            pl.BlockSpec((1, BQ, D), lambda b, q: (b, q, 0)),
            pl.BlockSpec((1, P2, D), lambda b, q: (b, 0, 0)),
            pl.BlockSpec((1, D, P2), lambda b, q: (b, 0, 0)),
        ],
        out_specs=[
            pl.BlockSpec((1, BQ, K), lambda b, q: (b, q, 0)),
            pl.BlockSpec((1, BQ, K), lambda b, q: (b, q, 0)),
        ],
        out_shape=[
            jax.ShapeDtypeStruct((n, p1n, K), jnp.int32),
            jax.ShapeDtypeStruct((n, p1n, K), jnp.float32),
        ],
    )(p1, p2, jnp.transpose(p2, (0, 2, 1)))
    return idx, dists


# speedup vs baseline: 16.2182x; 1.2598x over previous
import jax
import jax.numpy as jnp
from jax.experimental import pallas as pl

K = 16
BQ = 256
P2 = 4096
D = 128


def _knn_body(p1_ref, p2_ref, p2t_ref, idx_ref, dist_ref):
    p1b = p1_ref[0]
    p2b = p2_ref[0]
    p2t = p2t_ref[0]

    inner = jax.lax.dot_general(
        p1b, p2b, (((1,), (1,)), ((), ())),
        preferred_element_type=jnp.float32)

    sq1 = jnp.sum(p1b * p1b, axis=1, keepdims=True)
    sq2 = jnp.sum(p2t * p2t, axis=0, keepdims=True)

    d = sq1 + sq2 - 2.0 * inner

    iota_f = jax.lax.broadcasted_iota(jnp.int32, (BQ, P2), 1).astype(jnp.float32)
    inf = jnp.float32(jnp.inf)
    for k in range(K):
        m = jnp.min(d, axis=1, keepdims=True)
        cand = jnp.where(d == m, iota_f, jnp.float32(P2))
        jf = jnp.min(cand, axis=1, keepdims=True)
        idx_ref[0, :, pl.ds(k, 1)] = jf.astype(jnp.int32)
        dist_ref[0, :, pl.ds(k, 1)] = m
        d = jnp.where(iota_f == jf, inf, d)


def kernel(p1, p2):
    n, p1n, d_ = p1.shape
    grid = (n, p1n // BQ)
    idx, dists = pl.pallas_call(
        _knn_body,
        grid=grid,
        in_specs=[
            pl.BlockSpec((1, BQ, D), lambda b, q: (b, q, 0)),
            pl.BlockSpec((1, P2, D), lambda b, q: (b, 0, 0)),
            pl.BlockSpec((1, D, P2), lambda b, q: (b, 0, 0)),
        ],
        out_specs=[
            pl.BlockSpec((1, BQ, K), lambda b, q: (b, q, 0)),
            pl.BlockSpec((1, BQ, K), lambda b, q: (b, q, 0)),
        ],
        out_shape=[
            jax.ShapeDtypeStruct((n, p1n, K), jnp.int32),
            jax.ShapeDtypeStruct((n, p1n, K), jnp.float32),
        ],
    )(p1, p2, jnp.transpose(p2, (0, 2, 1)))
    return idx, dists


# speedup vs baseline: 20.8075x; 1.2830x over previous
import functools

import jax
import jax.numpy as jnp
from jax import lax
from jax.experimental import pallas as pl
from jax.experimental.pallas import tpu as pltpu
import jax.experimental.pallas.tpu_sc as plsc

K = 16
BQ = 256
P2 = 4096
D = 128
NQUAD = 1024
NCAND = 64
N = 4
NQ = N * P2
NWORKERS = 32
QPW = NQ // NWORKERS
CH = 8
BQ2 = 2048


def _tc_a_body(p1_ref, p2_ref, p2t_ref, d_ref, aidx_ref):
    p1b = p1_ref[0]
    p2b = p2_ref[0]
    p2t = p2t_ref[0]

    inner = lax.dot_general(
        p1b, p2b, (((1,), (1,)), ((), ())),
        preferred_element_type=jnp.float32)

    sq1 = jnp.sum(p1b * p1b, axis=1, keepdims=True)
    sq2 = jnp.sum(p2t * p2t, axis=0, keepdims=True)

    d = sq1 + sq2 - 2.0 * inner
    d_ref[0] = d

    qm = jnp.minimum(
        jnp.minimum(d[:, 0:1024], d[:, 1024:2048]),
        jnp.minimum(d[:, 2048:3072], d[:, 3072:4096]))

    iota_f = lax.broadcasted_iota(jnp.int32, (BQ, NQUAD), 1).astype(jnp.float32)
    inf = jnp.float32(jnp.inf)
    quads = []
    for _ in range(K):
        m = jnp.min(qm, axis=1, keepdims=True)
        cand = jnp.where(qm == m, iota_f, jnp.float32(NQUAD))
        jf = jnp.min(cand, axis=1, keepdims=True)
        quads.append(jf.astype(jnp.int32))
        qm = jnp.where(iota_f == jf, inf, qm)
    qcat = jnp.concatenate(quads, axis=1)

    row0 = pl.program_id(0) * P2 + pl.program_id(1) * BQ
    rows = row0 + lax.broadcasted_iota(jnp.int32, (BQ, 1), 0)
    cols = jnp.concatenate(
        [qcat + jnp.int32(s * NQUAD) for s in range(4)], axis=1)
    aidx_ref[0] = rows * jnp.int32(P2) + cols


def _sc_gather_body(d_hbm, aidx_hbm, vals_out, cidx, cval, sem):
    c = lax.axis_index("c")
    s = lax.axis_index("s")
    wid = c * 16 + s
    e0 = wid * (QPW * NCAND)

    def chunk(g, carry):
        base = e0 + g * (CH * NCAND)
        pltpu.sync_copy(aidx_hbm.at[pl.ds(base, CH * NCAND)], cidx)
        cps = [pltpu.async_copy(d_hbm.at[cidx.at[pl.ds(j * 128, 128)]],
                                cval.at[pl.ds(j * 128, 128)], sem)
               for j in range(4)]
        for cp in cps:
            cp.wait()
        pltpu.sync_copy(cval, vals_out.at[pl.ds(base, CH * NCAND)])
        return carry

    lax.fori_loop(0, QPW // CH, chunk, 0)


def _tc_c_body(vals_ref, aidx_ref, idx_ref, dist_ref):
    v = vals_ref[...]
    cols_f = (aidx_ref[...] & jnp.int32(P2 - 1)).astype(jnp.float32)
    inf = jnp.float32(jnp.inf)
    for k in range(K):
        m = jnp.min(v, axis=1, keepdims=True)
        cc = jnp.where(v == m, cols_f, jnp.float32(P2))
        cm = jnp.min(cc, axis=1, keepdims=True)
        idx_ref[:, pl.ds(k, 1)] = cm.astype(jnp.int32)
        dist_ref[:, pl.ds(k, 1)] = m
        v = jnp.where((v == m) & (cols_f == cm), inf, v)


def kernel(p1, p2):
    d_out, aidx = pl.pallas_call(
        _tc_a_body,
        grid=(N, P2 // BQ),
        in_specs=[
            pl.BlockSpec((1, BQ, D), lambda b, q: (b, q, 0)),
            pl.BlockSpec((1, P2, D), lambda b, q: (b, 0, 0)),
            pl.BlockSpec((1, D, P2), lambda b, q: (b, 0, 0)),
        ],
        out_specs=[
            pl.BlockSpec((1, BQ, P2), lambda b, q: (b, q, 0)),
            pl.BlockSpec((1, BQ, NCAND), lambda b, q: (b, q, 0)),
        ],
        out_shape=[
            jax.ShapeDtypeStruct((N, P2, P2), jnp.float32),
            jax.ShapeDtypeStruct((N, P2, NCAND), jnp.int32),
        ],
    )(p1, p2, jnp.transpose(p2, (0, 2, 1)))

    sc_gather = functools.partial(
        pl.kernel,
        out_type=jax.ShapeDtypeStruct((NQ * NCAND,), jnp.float32),
        mesh=plsc.VectorSubcoreMesh(core_axis_name="c", subcore_axis_name="s"),
        scratch_types=[
            pltpu.VMEM((CH * NCAND,), jnp.int32),
            pltpu.VMEM((CH * NCAND,), jnp.float32),
            pltpu.SemaphoreType.DMA,
        ],
    )(_sc_gather_body)
    vals = sc_gather(d_out.reshape(-1), aidx.reshape(-1))

    idx, dists = pl.pallas_call(
        _tc_c_body,
        grid=(NQ // BQ2,),
        in_specs=[
            pl.BlockSpec((BQ2, NCAND), lambda i: (i, 0)),
            pl.BlockSpec((BQ2, NCAND), lambda i: (i, 0)),
        ],
        out_specs=[
            pl.BlockSpec((BQ2, K), lambda i: (i, 0)),
            pl.BlockSpec((BQ2, K), lambda i: (i, 0)),
        ],
        out_shape=[
            jax.ShapeDtypeStruct((NQ, K), jnp.int32),
            jax.ShapeDtypeStruct((NQ, K), jnp.float32),
        ],
    )(vals.reshape(NQ, NCAND), aidx.reshape(NQ, NCAND))
    return idx.reshape(N, P2, K), dists.reshape(N, P2, K)


# speedup vs baseline: 25.1808x; 1.2102x over previous
import functools

import jax
import jax.numpy as jnp
from jax import lax
from jax.experimental import pallas as pl
from jax.experimental.pallas import tpu as pltpu
import jax.experimental.pallas.tpu_sc as plsc

K = 16
BQ = 256
P2 = 4096
D = 128
NQUAD = 1024
NCAND = 64
N = 4
NQ = N * P2
NWORKERS = 32
QPW = NQ // NWORKERS
CH = 8
BQ2 = 2048


def _tc_a_body(p1_ref, p2_ref, p2t_ref, d_ref, aidx_ref):
    p1b = p1_ref[0]
    p2b = p2_ref[0]
    p2t = p2t_ref[0]

    inner = lax.dot_general(
        p1b, p2b, (((1,), (1,)), ((), ())),
        preferred_element_type=jnp.float32)

    sq1 = jnp.sum(p1b * p1b, axis=1, keepdims=True)
    sq2 = jnp.sum(p2t * p2t, axis=0, keepdims=True)

    d = sq1 + sq2 - 2.0 * inner
    d_ref[...] = d.reshape(BQ * P2)

    qm = jnp.minimum(
        jnp.minimum(d[:, 0:1024], d[:, 1024:2048]),
        jnp.minimum(d[:, 2048:3072], d[:, 3072:4096]))

    iota_f = lax.broadcasted_iota(jnp.int32, (BQ, NQUAD), 1).astype(jnp.float32)
    inf = jnp.float32(jnp.inf)
    quads = []
    for _ in range(K):
        m = jnp.min(qm, axis=1, keepdims=True)
        cand = jnp.where(qm == m, iota_f, jnp.float32(NQUAD))
        jf = jnp.min(cand, axis=1, keepdims=True)
        quads.append(jf.astype(jnp.int32))
        qm = jnp.where(iota_f == jf, inf, qm)
    qcat = jnp.concatenate(quads, axis=1)

    row0 = pl.program_id(0) * P2 + pl.program_id(1) * BQ
    rows = row0 + lax.broadcasted_iota(jnp.int32, (BQ, 1), 0)
    cols = jnp.concatenate(
        [qcat + jnp.int32(s * NQUAD) for s in range(4)], axis=1)
    aidx_ref[0] = rows * jnp.int32(P2) + cols


def _sc_gather_body(d_hbm, aidx_hbm, vals_out, cidx, cval, sem):
    c = lax.axis_index("c")
    s = lax.axis_index("s")
    wid = c * 16 + s
    e0 = wid * (QPW * NCAND)

    def chunk(g, carry):
        base = e0 + g * (CH * NCAND)
        pltpu.sync_copy(aidx_hbm.at[pl.ds(base, CH * NCAND)], cidx)
        cps = [pltpu.async_copy(d_hbm.at[cidx.at[pl.ds(j * 128, 128)]],
                                cval.at[pl.ds(j * 128, 128)], sem)
               for j in range(4)]
        for cp in cps:
            cp.wait()
        pltpu.sync_copy(cval, vals_out.at[pl.ds(base, CH * NCAND)])
        return carry

    lax.fori_loop(0, QPW // CH, chunk, 0)


def _tc_c_body(vals_ref, aidx_ref, idx_ref, dist_ref):
    v = vals_ref[...]
    cols_f = (aidx_ref[...] & jnp.int32(P2 - 1)).astype(jnp.float32)
    inf = jnp.float32(jnp.inf)
    for k in range(K):
        m = jnp.min(v, axis=1, keepdims=True)
        cc = jnp.where(v == m, cols_f, jnp.float32(P2))
        cm = jnp.min(cc, axis=1, keepdims=True)
        idx_ref[:, pl.ds(k, 1)] = cm.astype(jnp.int32)
        dist_ref[:, pl.ds(k, 1)] = m
        v = jnp.where((v == m) & (cols_f == cm), inf, v)


def kernel(p1, p2):
    d_out, aidx = pl.pallas_call(
        _tc_a_body,
        grid=(N, P2 // BQ),
        in_specs=[
            pl.BlockSpec((1, BQ, D), lambda b, q: (b, q, 0)),
            pl.BlockSpec((1, P2, D), lambda b, q: (b, 0, 0)),
            pl.BlockSpec((1, D, P2), lambda b, q: (b, 0, 0)),
        ],
        out_specs=[
            pl.BlockSpec((BQ * P2,), lambda b, q: (b * (P2 // BQ) + q,)),
            pl.BlockSpec((1, BQ, NCAND), lambda b, q: (b, q, 0)),
        ],
        out_shape=[
            jax.ShapeDtypeStruct((N * P2 * P2,), jnp.float32),
            jax.ShapeDtypeStruct((N, P2, NCAND), jnp.int32),
        ],
    )(p1, p2, jnp.transpose(p2, (0, 2, 1)))

    sc_gather = functools.partial(
        pl.kernel,
        out_type=jax.ShapeDtypeStruct((NQ * NCAND,), jnp.float32),
        mesh=plsc.VectorSubcoreMesh(core_axis_name="c", subcore_axis_name="s"),
        scratch_types=[
            pltpu.VMEM((CH * NCAND,), jnp.int32),
            pltpu.VMEM((CH * NCAND,), jnp.float32),
            pltpu.SemaphoreType.DMA,
        ],
    )(_sc_gather_body)
    vals = sc_gather(d_out, aidx.reshape(-1))

    idx, dists = pl.pallas_call(
        _tc_c_body,
        grid=(NQ // BQ2,),
        in_specs=[
            pl.BlockSpec((BQ2, NCAND), lambda i: (i, 0)),
            pl.BlockSpec((BQ2, NCAND), lambda i: (i, 0)),
        ],
        out_specs=[
            pl.BlockSpec((BQ2, K), lambda i: (i, 0)),
            pl.BlockSpec((BQ2, K), lambda i: (i, 0)),
        ],
        out_shape=[
            jax.ShapeDtypeStruct((NQ, K), jnp.int32),
            jax.ShapeDtypeStruct((NQ, K), jnp.float32),
        ],
    )(vals.reshape(NQ, NCAND), aidx.reshape(NQ, NCAND))
    return idx.reshape(N, P2, K), dists.reshape(N, P2, K)


# speedup vs baseline: 27.6236x; 1.0970x over previous
import functools

import jax
import jax.numpy as jnp
from jax import lax
from jax.experimental import pallas as pl
from jax.experimental.pallas import tpu as pltpu
import jax.experimental.pallas.tpu_sc as plsc

K = 16
BQ = 256
P2 = 4096
D = 128
NQUAD = 1024
NCAND = 64
N = 4
NQ = N * P2
NWORKERS = 32
QPW = NQ // NWORKERS
CH = 32
BQ2 = 2048


def _tc_a_body(p1_ref, p2_ref, p2t_ref, d_ref, aidx_ref):
    p1b = p1_ref[0]
    p2b = p2_ref[0]
    p2t = p2t_ref[0]

    inner = lax.dot_general(
        p1b, p2b, (((1,), (1,)), ((), ())),
        preferred_element_type=jnp.float32)

    sq1 = jnp.sum(p1b * p1b, axis=1, keepdims=True)
    sq2 = jnp.sum(p2t * p2t, axis=0, keepdims=True)

    d = sq1 + sq2 - 2.0 * inner
    d_ref[...] = d.reshape(BQ * P2)

    qm = jnp.minimum(
        jnp.minimum(d[:, 0:1024], d[:, 1024:2048]),
        jnp.minimum(d[:, 2048:3072], d[:, 3072:4096]))

    iota_f = lax.broadcasted_iota(jnp.int32, (BQ, NQUAD), 1).astype(jnp.float32)
    inf = jnp.float32(jnp.inf)
    quads = []
    for _ in range(K):
        m = jnp.min(qm, axis=1, keepdims=True)
        cand = jnp.where(qm == m, iota_f, jnp.float32(NQUAD))
        jf = jnp.min(cand, axis=1, keepdims=True)
        quads.append(jf.astype(jnp.int32))
        qm = jnp.where(iota_f == jf, inf, qm)
    qcat = jnp.concatenate(quads, axis=1)

    row0 = pl.program_id(0) * P2 + pl.program_id(1) * BQ
    rows = row0 + lax.broadcasted_iota(jnp.int32, (BQ, 1), 0)
    cols = jnp.concatenate(
        [qcat + jnp.int32(s * NQUAD) for s in range(4)], axis=1)
    aidx_ref[0] = rows * jnp.int32(P2) + cols


def _sc_gather_body(d_hbm, aidx_hbm, vals_out, cidx, cval, sem):
    c = lax.axis_index("c")
    s = lax.axis_index("s")
    wid = c * 16 + s
    e0 = wid * (QPW * NCAND)

    def chunk(g, carry):
        base = e0 + g * (CH * NCAND)
        pltpu.sync_copy(aidx_hbm.at[pl.ds(base, CH * NCAND)], cidx)
        cps = [pltpu.async_copy(d_hbm.at[cidx.at[pl.ds(j * 128, 128)]],
                                cval.at[pl.ds(j * 128, 128)], sem)
               for j in range(CH * NCAND // 128)]
        for cp in cps:
            cp.wait()
        pltpu.sync_copy(cval, vals_out.at[pl.ds(base, CH * NCAND)])
        return carry

    lax.fori_loop(0, QPW // CH, chunk, 0)


def _tc_c_body(vals_ref, aidx_ref, idx_ref, dist_ref):
    v = vals_ref[...]
    cols_f = (aidx_ref[...] & jnp.int32(P2 - 1)).astype(jnp.float32)
    inf = jnp.float32(jnp.inf)
    for k in range(K):
        m = jnp.min(v, axis=1, keepdims=True)
        cc = jnp.where(v == m, cols_f, jnp.float32(P2))
        cm = jnp.min(cc, axis=1, keepdims=True)
        idx_ref[:, pl.ds(k, 1)] = cm.astype(jnp.int32)
        dist_ref[:, pl.ds(k, 1)] = m
        v = jnp.where((v == m) & (cols_f == cm), inf, v)


def kernel(p1, p2):
    d_out, aidx = pl.pallas_call(
        _tc_a_body,
        grid=(N, P2 // BQ),
        in_specs=[
            pl.BlockSpec((1, BQ, D), lambda b, q: (b, q, 0)),
            pl.BlockSpec((1, P2, D), lambda b, q: (b, 0, 0)),
            pl.BlockSpec((1, D, P2), lambda b, q: (b, 0, 0)),
        ],
        out_specs=[
            pl.BlockSpec((BQ * P2,), lambda b, q: (b * (P2 // BQ) + q,)),
            pl.BlockSpec((1, BQ, NCAND), lambda b, q: (b, q, 0)),
        ],
        out_shape=[
            jax.ShapeDtypeStruct((N * P2 * P2,), jnp.float32),
            jax.ShapeDtypeStruct((N, P2, NCAND), jnp.int32),
        ],
    )(p1, p2, jnp.transpose(p2, (0, 2, 1)))

    sc_gather = functools.partial(
        pl.kernel,
        out_type=jax.ShapeDtypeStruct((NQ * NCAND,), jnp.float32),
        mesh=plsc.VectorSubcoreMesh(core_axis_name="c", subcore_axis_name="s"),
        scratch_types=[
            pltpu.VMEM((CH * NCAND,), jnp.int32),
            pltpu.VMEM((CH * NCAND,), jnp.float32),
            pltpu.SemaphoreType.DMA,
        ],
    )(_sc_gather_body)
    vals = sc_gather(d_out, aidx.reshape(-1))

    idx, dists = pl.pallas_call(
        _tc_c_body,
        grid=(NQ // BQ2,),
        in_specs=[
            pl.BlockSpec((BQ2, NCAND), lambda i: (i, 0)),
            pl.BlockSpec((BQ2, NCAND), lambda i: (i, 0)),
        ],
        out_specs=[
            pl.BlockSpec((BQ2, K), lambda i: (i, 0)),
            pl.BlockSpec((BQ2, K), lambda i: (i, 0)),
        ],
        out_shape=[
            jax.ShapeDtypeStruct((NQ, K), jnp.int32),
            jax.ShapeDtypeStruct((NQ, K), jnp.float32),
        ],
    )(vals.reshape(NQ, NCAND), aidx.reshape(NQ, NCAND))
    return idx.reshape(N, P2, K), dists.reshape(N, P2, K)


# speedup vs baseline: 28.8888x; 1.0458x over previous
import functools

import jax
import jax.numpy as jnp
from jax import lax
from jax.experimental import pallas as pl
from jax.experimental.pallas import tpu as pltpu
import jax.experimental.pallas.tpu_sc as plsc

K = 16
BQ = 512
P2 = 4096
D = 128
NQUAD = 1024
NCAND = 64
N = 4
NQ = N * P2
NWORKERS = 32
QPW = NQ // NWORKERS
CH = 64
BQ2 = 2048


def _tc_a_body(p1_ref, p2_ref, p2t_ref, d_ref, aidx_ref):
    p1b = p1_ref[0]
    p2b = p2_ref[0]
    p2t = p2t_ref[0]

    inner = lax.dot_general(
        p1b, p2b, (((1,), (1,)), ((), ())),
        preferred_element_type=jnp.float32)

    sq1 = jnp.sum(p1b * p1b, axis=1, keepdims=True)
    sq2 = jnp.sum(p2t * p2t, axis=0, keepdims=True)

    d = sq1 + sq2 - 2.0 * inner
    d_ref[...] = d.reshape(BQ * P2)

    qm = jnp.minimum(
        jnp.minimum(d[:, 0:1024], d[:, 1024:2048]),
        jnp.minimum(d[:, 2048:3072], d[:, 3072:4096]))

    iota_f = lax.broadcasted_iota(jnp.int32, (BQ, NQUAD), 1).astype(jnp.float32)
    inf = jnp.float32(jnp.inf)
    quads = []
    for _ in range(K):
        m = jnp.min(qm, axis=1, keepdims=True)
        cand = jnp.where(qm == m, iota_f, jnp.float32(NQUAD))
        jf = jnp.min(cand, axis=1, keepdims=True)
        quads.append(jf.astype(jnp.int32))
        qm = jnp.where(iota_f == jf, inf, qm)
    qcat = jnp.concatenate(quads, axis=1)

    row0 = pl.program_id(0) * P2 + pl.program_id(1) * BQ
    rows = row0 + lax.broadcasted_iota(jnp.int32, (BQ, 1), 0)
    cols = jnp.concatenate(
        [qcat + jnp.int32(s * NQUAD) for s in range(4)], axis=1)
    aidx_ref[0] = rows * jnp.int32(P2) + cols


def _sc_gather_body(d_hbm, aidx_hbm, vals_out, cidx, cval, sem):
    c = lax.axis_index("c")
    s = lax.axis_index("s")
    wid = c * 16 + s
    e0 = wid * (QPW * NCAND)

    def chunk(g, carry):
        base = e0 + g * (CH * NCAND)
        pltpu.sync_copy(aidx_hbm.at[pl.ds(base, CH * NCAND)], cidx)
        cps = [pltpu.async_copy(d_hbm.at[cidx.at[pl.ds(j * 128, 128)]],
                                cval.at[pl.ds(j * 128, 128)], sem)
               for j in range(CH * NCAND // 128)]
        for cp in cps:
            cp.wait()
        pltpu.sync_copy(cval, vals_out.at[pl.ds(base, CH * NCAND)])
        return carry

    lax.fori_loop(0, QPW // CH, chunk, 0)


def _tc_c_body(vals_ref, aidx_ref, idx_ref, dist_ref):
    v = vals_ref[...]
    cols_f = (aidx_ref[...] & jnp.int32(P2 - 1)).astype(jnp.float32)
    inf = jnp.float32(jnp.inf)
    for k in range(K):
        m = jnp.min(v, axis=1, keepdims=True)
        cc = jnp.where(v == m, cols_f, jnp.float32(P2))
        cm = jnp.min(cc, axis=1, keepdims=True)
        idx_ref[:, pl.ds(k, 1)] = cm.astype(jnp.int32)
        dist_ref[:, pl.ds(k, 1)] = m
        v = jnp.where((v == m) & (cols_f == cm), inf, v)


def kernel(p1, p2):
    d_out, aidx = pl.pallas_call(
        _tc_a_body,
        grid=(N, P2 // BQ),
        in_specs=[
            pl.BlockSpec((1, BQ, D), lambda b, q: (b, q, 0)),
            pl.BlockSpec((1, P2, D), lambda b, q: (b, 0, 0)),
            pl.BlockSpec((1, D, P2), lambda b, q: (b, 0, 0)),
        ],
        out_specs=[
            pl.BlockSpec((BQ * P2,), lambda b, q: (b * (P2 // BQ) + q,)),
            pl.BlockSpec((1, BQ, NCAND), lambda b, q: (b, q, 0)),
        ],
        out_shape=[
            jax.ShapeDtypeStruct((N * P2 * P2,), jnp.float32),
            jax.ShapeDtypeStruct((N, P2, NCAND), jnp.int32),
        ],
    )(p1, p2, jnp.transpose(p2, (0, 2, 1)))

    sc_gather = functools.partial(
        pl.kernel,
        out_type=jax.ShapeDtypeStruct((NQ * NCAND,), jnp.float32),
        mesh=plsc.VectorSubcoreMesh(core_axis_name="c", subcore_axis_name="s"),
        scratch_types=[
            pltpu.VMEM((CH * NCAND,), jnp.int32),
            pltpu.VMEM((CH * NCAND,), jnp.float32),
            pltpu.SemaphoreType.DMA,
        ],
    )(_sc_gather_body)
    vals = sc_gather(d_out, aidx.reshape(-1))

    idx, dists = pl.pallas_call(
        _tc_c_body,
        grid=(NQ // BQ2,),
        in_specs=[
            pl.BlockSpec((BQ2, NCAND), lambda i: (i, 0)),
            pl.BlockSpec((BQ2, NCAND), lambda i: (i, 0)),
        ],
        out_specs=[
            pl.BlockSpec((BQ2, K), lambda i: (i, 0)),
            pl.BlockSpec((BQ2, K), lambda i: (i, 0)),
        ],
        out_shape=[
            jax.ShapeDtypeStruct((NQ, K), jnp.int32),
            jax.ShapeDtypeStruct((NQ, K), jnp.float32),
        ],
    )(vals.reshape(NQ, NCAND), aidx.reshape(NQ, NCAND))
    return idx.reshape(N, P2, K), dists.reshape(N, P2, K)


# speedup vs baseline: 31.9658x; 1.1065x over previous
import functools

import jax
import jax.numpy as jnp
from jax import lax
from jax.experimental import pallas as pl
from jax.experimental.pallas import tpu as pltpu
import jax.experimental.pallas.tpu_sc as plsc

K = 16
BQ = 512
P2 = 4096
D = 128
NQUAD = 512
NCAND = 128
N = 4
NQ = N * P2
NWORKERS = 32
QPW = NQ // NWORKERS
CH = 32
BQ2 = 2048


def _tc_a_body(p1_ref, p2_ref, p2t_ref, d_ref, aidx_ref):
    p1b = p1_ref[0]
    p2b = p2_ref[0]
    p2t = p2t_ref[0]

    inner2 = lax.dot_general(
        p1b * jnp.float32(-2.0), p2b, (((1,), (1,)), ((), ())),
        preferred_element_type=jnp.float32)

    sq1 = jnp.sum(p1b * p1b, axis=1, keepdims=True)
    sq2 = jnp.sum(p2t * p2t, axis=0, keepdims=True)

    d = (sq1 + sq2) + inner2
    d_ref[...] = d.reshape(BQ * P2)

    qm = d[:, 0:NQUAD]
    for s_ in range(1, 8):
        qm = jnp.minimum(qm, d[:, s_ * NQUAD:(s_ + 1) * NQUAD])

    iota_f = lax.broadcasted_iota(jnp.int32, (BQ, NQUAD), 1).astype(jnp.float32)
    inf = jnp.float32(jnp.inf)
    quads = []
    for _ in range(K):
        m = jnp.min(qm, axis=1, keepdims=True)
        cand = jnp.where(qm == m, iota_f, jnp.float32(NQUAD))
        jf = jnp.min(cand, axis=1, keepdims=True)
        quads.append(jf.astype(jnp.int32))
        qm = jnp.where(iota_f == jf, inf, qm)
    qcat = jnp.concatenate(quads, axis=1)

    row0 = pl.program_id(0) * P2 + pl.program_id(1) * BQ
    rows = row0 + lax.broadcasted_iota(jnp.int32, (BQ, 1), 0)
    cols = jnp.concatenate(
        [qcat + jnp.int32(s * NQUAD) for s in range(8)], axis=1)
    aidx_ref[0] = rows * jnp.int32(P2) + cols


def _sc_gather_body(d_hbm, aidx_hbm, vals_out, cidx, cval, sem):
    c = lax.axis_index("c")
    s = lax.axis_index("s")
    wid = c * 16 + s
    e0 = wid * (QPW * NCAND)

    def chunk(g, carry):
        base = e0 + g * (CH * NCAND)
        pltpu.sync_copy(aidx_hbm.at[pl.ds(base, CH * NCAND)], cidx)
        cps = [pltpu.async_copy(d_hbm.at[cidx.at[pl.ds(j * 128, 128)]],
                                cval.at[pl.ds(j * 128, 128)], sem)
               for j in range(CH * NCAND // 128)]
        for cp in cps:
            cp.wait()
        pltpu.sync_copy(cval, vals_out.at[pl.ds(base, CH * NCAND)])
        return carry

    lax.fori_loop(0, QPW // CH, chunk, 0)


def _tc_c_body(vals_ref, aidx_ref, idx_ref, dist_ref):
    v = vals_ref[...]
    cols_f = (aidx_ref[...] & jnp.int32(P2 - 1)).astype(jnp.float32)
    inf = jnp.float32(jnp.inf)
    for k in range(K):
        m = jnp.min(v, axis=1, keepdims=True)
        cc = jnp.where(v == m, cols_f, jnp.float32(P2))
        cm = jnp.min(cc, axis=1, keepdims=True)
        idx_ref[:, pl.ds(k, 1)] = cm.astype(jnp.int32)
        dist_ref[:, pl.ds(k, 1)] = m
        v = jnp.where((v == m) & (cols_f == cm), inf, v)


def kernel(p1, p2):
    d_out, aidx = pl.pallas_call(
        _tc_a_body,
        grid=(N, P2 // BQ),
        in_specs=[
            pl.BlockSpec((1, BQ, D), lambda b, q: (b, q, 0)),
            pl.BlockSpec((1, P2, D), lambda b, q: (b, 0, 0)),
            pl.BlockSpec((1, D, P2), lambda b, q: (b, 0, 0)),
        ],
        out_specs=[
            pl.BlockSpec((BQ * P2,), lambda b, q: (b * (P2 // BQ) + q,)),
            pl.BlockSpec((1, BQ, NCAND), lambda b, q: (b, q, 0)),
        ],
        out_shape=[
            jax.ShapeDtypeStruct((N * P2 * P2,), jnp.float32),
            jax.ShapeDtypeStruct((N, P2, NCAND), jnp.int32),
        ],
    )(p1, p2, jnp.transpose(p2, (0, 2, 1)))

    sc_gather = functools.partial(
        pl.kernel,
        out_type=jax.ShapeDtypeStruct((NQ * NCAND,), jnp.float32),
        mesh=plsc.VectorSubcoreMesh(core_axis_name="c", subcore_axis_name="s"),
        scratch_types=[
            pltpu.VMEM((CH * NCAND,), jnp.int32),
            pltpu.VMEM((CH * NCAND,), jnp.float32),
            pltpu.SemaphoreType.DMA,
        ],
    )(_sc_gather_body)
    vals = sc_gather(d_out, aidx.reshape(-1))

    idx, dists = pl.pallas_call(
        _tc_c_body,
        grid=(NQ // BQ2,),
        in_specs=[
            pl.BlockSpec((BQ2, NCAND), lambda i: (i, 0)),
            pl.BlockSpec((BQ2, NCAND), lambda i: (i, 0)),
        ],
        out_specs=[
            pl.BlockSpec((BQ2, K), lambda i: (i, 0)),
            pl.BlockSpec((BQ2, K), lambda i: (i, 0)),
        ],
        out_shape=[
            jax.ShapeDtypeStruct((NQ, K), jnp.int32),
            jax.ShapeDtypeStruct((NQ, K), jnp.float32),
        ],
    )(vals.reshape(NQ, NCAND), aidx.reshape(NQ, NCAND))
    return idx.reshape(N, P2, K), dists.reshape(N, P2, K)


# speedup vs baseline: 33.7707x; 1.0565x over previous
import functools

import jax
import jax.numpy as jnp
from jax import lax
from jax.experimental import pallas as pl
from jax.experimental.pallas import tpu as pltpu
import jax.experimental.pallas.tpu_sc as plsc

K = 16
BQ = 512
P2 = 4096
D = 128
NQUAD = 512
NCAND = 128
N = 4
NQ = N * P2
NWORKERS = 32
QPW = NQ // NWORKERS
CH = 32
BQ2 = 2048


def _tc_a_body(p1_ref, p2_ref, p2t_ref, d_ref, aidx_ref):
    p1b = p1_ref[0]
    p2b = p2_ref[0]
    p2t = p2t_ref[0]

    inner2 = lax.dot_general(
        p1b * jnp.float32(-2.0), p2b, (((1,), (1,)), ((), ())),
        preferred_element_type=jnp.float32)

    sq1 = jnp.sum(p1b * p1b, axis=1, keepdims=True)
    sq2 = jnp.sum(p2t * p2t, axis=0, keepdims=True)

    d = (sq1 + sq2) + inner2
    d_ref[...] = d.reshape(BQ * P2)

    qm = d[:, 0:NQUAD]
    for s_ in range(1, 8):
        qm = jnp.minimum(qm, d[:, s_ * NQUAD:(s_ + 1) * NQUAD])

    iota_f = lax.broadcasted_iota(jnp.int32, (BQ, NQUAD), 1).astype(jnp.float32)
    inf = jnp.float32(jnp.inf)
    quads = []
    for _ in range(K):
        m = jnp.min(qm, axis=1, keepdims=True)
        cand = jnp.where(qm == m, iota_f, jnp.float32(NQUAD))
        jf = jnp.min(cand, axis=1, keepdims=True)
        quads.append(jf.astype(jnp.int32))
        qm = jnp.where(iota_f == jf, inf, qm)
    qcat = jnp.concatenate(quads, axis=1)

    row0 = pl.program_id(0) * P2 + pl.program_id(1) * BQ
    rows = row0 + lax.broadcasted_iota(jnp.int32, (BQ, 1), 0)
    cols = jnp.concatenate(
        [qcat + jnp.int32(s * NQUAD) for s in range(8)], axis=1)
    aidx_ref[0] = rows * jnp.int32(P2) + cols


def _sc_gather_body(nq_span, d_hbm, aidx_hbm, vals_out, cidx, cval, sem):
    c = lax.axis_index("c")
    s = lax.axis_index("s")
    wid = c * 16 + s
    qpw = nq_span // NWORKERS
    e0 = wid * (qpw * NCAND)

    def chunk(g, carry):
        base = e0 + g * (CH * NCAND)
        pltpu.sync_copy(aidx_hbm.at[pl.ds(base, CH * NCAND)], cidx)
        cps = [pltpu.async_copy(d_hbm.at[cidx.at[pl.ds(j * 128, 128)]],
                                cval.at[pl.ds(j * 128, 128)], sem)
               for j in range(CH * NCAND // 128)]
        for cp in cps:
            cp.wait()
        pltpu.sync_copy(cval, vals_out.at[pl.ds(base, CH * NCAND)])
        return carry

    lax.fori_loop(0, qpw // CH, chunk, 0)


def _tc_c_body(vals_ref, aidx_ref, idx_ref, dist_ref):
    v = vals_ref[...]
    cols_f = (aidx_ref[...] & jnp.int32(P2 - 1)).astype(jnp.float32)
    inf = jnp.float32(jnp.inf)
    for k in range(K):
        m = jnp.min(v, axis=1, keepdims=True)
        cc = jnp.where(v == m, cols_f, jnp.float32(P2))
        cm = jnp.min(cc, axis=1, keepdims=True)
        idx_ref[:, pl.ds(k, 1)] = cm.astype(jnp.int32)
        dist_ref[:, pl.ds(k, 1)] = m
        v = jnp.where((v == m) & (cols_f == cm), inf, v)


def kernel(p1, p2):
    d_out, aidx = pl.pallas_call(
        _tc_a_body,
        grid=(N, P2 // BQ),
        in_specs=[
            pl.BlockSpec((1, BQ, D), lambda b, q: (b, q, 0)),
            pl.BlockSpec((1, P2, D), lambda b, q: (b, 0, 0)),
            pl.BlockSpec((1, D, P2), lambda b, q: (b, 0, 0)),
        ],
        out_specs=[
            pl.BlockSpec((BQ * P2,), lambda b, q: (b * (P2 // BQ) + q,)),
            pl.BlockSpec((1, BQ, NCAND), lambda b, q: (b, q, 0)),
        ],
        out_shape=[
            jax.ShapeDtypeStruct((N * P2 * P2,), jnp.float32),
            jax.ShapeDtypeStruct((N, P2, NCAND), jnp.int32),
        ],
    )(p1, p2, jnp.transpose(p2, (0, 2, 1)))

    HQ = NQ // 2
    sc_gather = functools.partial(
        pl.kernel,
        out_type=jax.ShapeDtypeStruct((HQ * NCAND,), jnp.float32),
        mesh=plsc.VectorSubcoreMesh(core_axis_name="c", subcore_axis_name="s"),
        scratch_types=[
            pltpu.VMEM((CH * NCAND,), jnp.int32),
            pltpu.VMEM((CH * NCAND,), jnp.float32),
            pltpu.SemaphoreType.DMA,
        ],
    )(functools.partial(_sc_gather_body, HQ))

    aidx_flat = aidx.reshape(-1)
    aidx_2d = aidx.reshape(NQ, NCAND)
    idx_halves, dist_halves = [], []
    for h in range(2):
        a_h = lax.slice(aidx_flat, (h * HQ * NCAND,), ((h + 1) * HQ * NCAND,))
        vals_h = sc_gather(d_out, a_h)
        idx_h, dists_h = pl.pallas_call(
            _tc_c_body,
            grid=(HQ // BQ2,),
            in_specs=[
                pl.BlockSpec((BQ2, NCAND), lambda i: (i, 0)),
                pl.BlockSpec((BQ2, NCAND), lambda i: (i, 0)),
            ],
            out_specs=[
                pl.BlockSpec((BQ2, K), lambda i: (i, 0)),
                pl.BlockSpec((BQ2, K), lambda i: (i, 0)),
            ],
            out_shape=[
                jax.ShapeDtypeStruct((HQ, K), jnp.int32),
                jax.ShapeDtypeStruct((HQ, K), jnp.float32),
            ],
        )(vals_h.reshape(HQ, NCAND), lax.slice(aidx_2d, (h * HQ, 0), ((h + 1) * HQ, NCAND)))
        idx_halves.append(idx_h)
        dist_halves.append(dists_h)
    idx = jnp.concatenate(idx_halves, axis=0)
    dists = jnp.concatenate(dist_halves, axis=0)
    return idx.reshape(N, P2, K), dists.reshape(N, P2, K)
